# baseline (speedup 1.0000x reference)
import functools

import jax
import jax.numpy as jnp
from jax.experimental import pallas as pl
from jax.experimental.pallas import tpu as pltpu

HID = 64
HEADS = 16
INTEREST = 3
N_NODE = 100000
B = 256
NEG = 0.2
H3 = INTEREST * HID

ZBLK = 2048


def _zmax_body(sh_ref, emb_ref, out_ref):
    e = emb_ref[...]
    dn = (((1,), (1,)), ((), ()))
    z0 = jax.lax.dot_general(sh_ref[0], e, dn, preferred_element_type=jnp.float32)
    z1 = jax.lax.dot_general(sh_ref[1], e, dn, preferred_element_type=jnp.float32)
    z2 = jax.lax.dot_general(sh_ref[2], e, dn, preferred_element_type=jnp.float32)
    out_ref[...] = jnp.maximum(jnp.maximum(z0, z1), z2)


def _zmax(sh3, emb):
    grid = (pl.cdiv(N_NODE, ZBLK),)
    return pl.pallas_call(
        _zmax_body,
        grid=grid,
        in_specs=[
            pl.BlockSpec((INTEREST, B, HID), lambda j: (0, 0, 0)),
            pl.BlockSpec((ZBLK, HID), lambda j: (j, 0)),
        ],
        out_specs=pl.BlockSpec((B, ZBLK), lambda j: (0, j)),
        out_shape=jax.ShapeDtypeStruct((B, N_NODE), jnp.float32),
    )(sh3, emb)


def _gat(x, src, dst, emask, W, a_src, a_dst, b, heads, outc):
    n = x.shape[0]
    h = (x @ W).reshape(n, heads, outc)
    asrc = jnp.sum(h * a_src, axis=-1)
    adst = jnp.sum(h * a_dst, axis=-1)
    logit = asrc[src] + adst[dst]
    logit = jnp.where(logit >= 0, logit, NEG * logit)
    logit = jnp.where(emask[:, None], logit, -1e9)
    m = jax.ops.segment_max(logit, dst, num_segments=n)
    pexp = jnp.exp(logit - m[dst])
    pexp = jnp.where(emask[:, None], pexp, 0.0)
    den = jax.ops.segment_sum(pexp, dst, num_segments=n)
    alpha = pexp / jnp.clip(den[dst], 1e-16, None)
    out = jax.ops.segment_sum(h[src] * alpha[:, :, None], dst, num_segments=n)
    return out.reshape(n, heads * outc) + b


def kernel(x, edge_index, batch, edge_attr, params):
    p = params
    xi = x - 1
    emb = p['emb'][xi]
    g = jax.nn.softmax((emb @ p['w_int'].T) / 0.1, axis=-1)
    mask = (g > 1.0 / INTEREST).T
    src0 = edge_index[0]
    dst0 = edge_index[1]
    n = emb.shape[0]
    loop = jnp.arange(n)
    src = jnp.concatenate([src0, loop])
    dst = jnp.concatenate([dst0, loop])
    notself = src0 != dst0
    ones_n = jnp.ones((n,), dtype=bool)
    hiddens = []
    for i in range(INTEREST):
        emask = jnp.concatenate([mask[i][src0] & mask[i][dst0] & notself, ones_n])
        xin = g[:, i:i + 1] * emb
        h1 = jax.nn.relu(_gat(xin, src, dst, emask, p['g%da_W' % i], p['g%da_as' % i], p['g%da_ad' % i], p['g%da_b' % i], HEADS, HID))
        h2 = _gat(h1, src, dst, emask, p['g%db_W' % i], p['g%db_as' % i], p['g%db_ad' % i], p['g%db_b' % i], 1, HID)
        hiddens.append(h2)
    sess = jnp.concatenate(hiddens, axis=1)
    sections = jnp.bincount(batch, length=B)
    last_idx = jnp.cumsum(sections) - 1
    v_n = sess[last_idx]
    q1 = v_n[batch] @ p['W1'] + p['b1']
    q2 = sess @ p['W2'] + p['b2']
    a = jax.nn.sigmoid(q1 + q2) @ p['qw'] + p['qb']
    s_g = jax.ops.segment_sum(a * sess, batch, num_segments=B)
    s_h = jnp.concatenate([v_n, s_g], axis=1) @ p['W3'] + p['b3']
    sh3 = s_h.reshape(B, INTEREST, HID).transpose(1, 0, 2)
    return _zmax(sh3, p['emb'])


# speedup vs baseline: 2.3197x; 2.3197x over previous
import functools

import jax
import jax.numpy as jnp
from jax.experimental import pallas as pl
from jax.experimental.pallas import tpu as pltpu

HID = 64
HEADS = 16
INTEREST = 3
N_NODE = 100000
B = 256
NEG = 0.2
H3 = INTEREST * HID

ZBLK = 2048


def _zmax_body(sh_ref, emb_ref, out_ref):
    e = emb_ref[...]
    dn = (((1,), (1,)), ((), ()))
    z0 = jax.lax.dot_general(sh_ref[0], e, dn, preferred_element_type=jnp.float32)
    z1 = jax.lax.dot_general(sh_ref[1], e, dn, preferred_element_type=jnp.float32)
    z2 = jax.lax.dot_general(sh_ref[2], e, dn, preferred_element_type=jnp.float32)
    out_ref[...] = jnp.maximum(jnp.maximum(z0, z1), z2)


def _zmax(sh3, emb):
    grid = (pl.cdiv(N_NODE, ZBLK),)
    return pl.pallas_call(
        _zmax_body,
        grid=grid,
        in_specs=[
            pl.BlockSpec((INTEREST, B, HID), lambda j: (0, 0, 0)),
            pl.BlockSpec((ZBLK, HID), lambda j: (j, 0)),
        ],
        out_specs=pl.BlockSpec((B, ZBLK), lambda j: (0, j)),
        out_shape=jax.ShapeDtypeStruct((B, N_NODE), jnp.float32),
    )(sh3, emb)


def _gat(x, src, dst, emask, W, a_src, a_dst, b, heads, outc):
    n = x.shape[0]
    h = (x @ W).reshape(n, heads, outc)
    asrc = jnp.sum(h * a_src, axis=-1)
    adst = jnp.sum(h * a_dst, axis=-1)
    logit = asrc[src] + adst[dst]
    logit = jnp.where(logit >= 0, logit, NEG * logit)
    logit = jnp.where(emask[:, None], logit, -1e9)
    m = jax.ops.segment_max(logit, dst, num_segments=n)
    pexp = jnp.exp(logit - m[dst])
    pexp = jnp.where(emask[:, None], pexp, 0.0)
    den = jax.ops.segment_sum(pexp, dst, num_segments=n)
    alpha = pexp / jnp.clip(den[dst], 1e-16, None)
    msg = (h[src] * alpha[:, :, None]).reshape(-1, heads * outc)
    out = jax.ops.segment_sum(msg, dst, num_segments=n)
    return out + b


def kernel(x, edge_index, batch, edge_attr, params):
    p = params
    xi = x - 1
    emb = p['emb'][xi]
    g = jax.nn.softmax((emb @ p['w_int'].T) / 0.1, axis=-1)
    mask = (g > 1.0 / INTEREST).T
    src0 = edge_index[0]
    dst0 = edge_index[1]
    n = emb.shape[0]
    loop = jnp.arange(n)
    src = jnp.concatenate([src0, loop])
    dst = jnp.concatenate([dst0, loop])
    notself = src0 != dst0
    ones_n = jnp.ones((n,), dtype=bool)
    hiddens = []
    for i in range(INTEREST):
        emask = jnp.concatenate([mask[i][src0] & mask[i][dst0] & notself, ones_n])
        xin = g[:, i:i + 1] * emb
        h1 = jax.nn.relu(_gat(xin, src, dst, emask, p['g%da_W' % i], p['g%da_as' % i], p['g%da_ad' % i], p['g%da_b' % i], HEADS, HID))
        h2 = _gat(h1, src, dst, emask, p['g%db_W' % i], p['g%db_as' % i], p['g%db_ad' % i], p['g%db_b' % i], 1, HID)
        hiddens.append(h2)
    sess = jnp.concatenate(hiddens, axis=1)
    sections = jnp.bincount(batch, length=B)
    last_idx = jnp.cumsum(sections) - 1
    v_n = sess[last_idx]
    q1 = v_n[batch] @ p['W1'] + p['b1']
    q2 = sess @ p['W2'] + p['b2']
    a = jax.nn.sigmoid(q1 + q2) @ p['qw'] + p['qb']
    s_g = jax.ops.segment_sum(a * sess, batch, num_segments=B)
    s_h = jnp.concatenate([v_n, s_g], axis=1) @ p['W3'] + p['b3']
    sh3 = s_h.reshape(B, INTEREST, HID).transpose(1, 0, 2)
    return _zmax(sh3, p['emb'])


# speedup vs baseline: 3.4442x; 1.4847x over previous
import functools

import jax
import jax.numpy as jnp
from jax.experimental import pallas as pl
from jax.experimental.pallas import tpu as pltpu

HID = 64
HEADS = 16
INTEREST = 3
N_NODE = 100000
B = 256
NEG = 0.2
H3 = INTEREST * HID

ZBLK = 2048


def _zmax_body(sh_ref, emb_ref, out_ref):
    e = emb_ref[...]
    dn = (((1,), (1,)), ((), ()))
    z0 = jax.lax.dot_general(sh_ref[0], e, dn, preferred_element_type=jnp.float32)
    z1 = jax.lax.dot_general(sh_ref[1], e, dn, preferred_element_type=jnp.float32)
    z2 = jax.lax.dot_general(sh_ref[2], e, dn, preferred_element_type=jnp.float32)
    out_ref[...] = jnp.maximum(jnp.maximum(z0, z1), z2)


def _zmax(sh3, emb):
    grid = (pl.cdiv(N_NODE, ZBLK),)
    return pl.pallas_call(
        _zmax_body,
        grid=grid,
        in_specs=[
            pl.BlockSpec((INTEREST, B, HID), lambda j: (0, 0, 0)),
            pl.BlockSpec((ZBLK, HID), lambda j: (j, 0)),
        ],
        out_specs=pl.BlockSpec((B, ZBLK), lambda j: (0, j)),
        out_shape=jax.ShapeDtypeStruct((B, N_NODE), jnp.float32),
    )(sh3, emb)


def _gat_batched(h_all, src, dst, emask_all, asrc_all, adst_all, b_all, heads, outc):
    n = h_all.shape[0]
    emask = jnp.repeat(emask_all, heads, axis=1)
    logit = asrc_all[src] + adst_all[dst]
    logit = jnp.where(logit >= 0, logit, NEG * logit)
    logit = jnp.where(emask, logit, -1e9)
    m = jax.ops.segment_max(logit, dst, num_segments=n)
    pexp = jnp.exp(logit - m[dst])
    pexp = jnp.where(emask, pexp, 0.0)
    den = jax.ops.segment_sum(pexp, dst, num_segments=n)
    alpha = pexp / jnp.clip(den[dst], 1e-16, None)
    hs = h_all[src].reshape(-1, alpha.shape[1], outc)
    msg = (hs * alpha[:, :, None]).reshape(-1, h_all.shape[1])
    out = jax.ops.segment_sum(msg, dst, num_segments=n)
    return out + b_all


def kernel(x, edge_index, batch, edge_attr, params):
    p = params
    xi = x - 1
    emb = p['emb'][xi]
    g = jax.nn.softmax((emb @ p['w_int'].T) / 0.1, axis=-1)
    mask = (g > 1.0 / INTEREST).T
    src0 = edge_index[0]
    dst0 = edge_index[1]
    n = emb.shape[0]
    loop = jnp.arange(n)
    src = jnp.concatenate([src0, loop])
    dst = jnp.concatenate([dst0, loop])
    notself = src0 != dst0
    ones_n = jnp.ones((n,), dtype=bool)
    emask_all = jnp.stack(
        [jnp.concatenate([mask[i][src0] & mask[i][dst0] & notself, ones_n])
         for i in range(INTEREST)], axis=1)

    ha_list, asrc_l, adst_l = [], [], []
    for i in range(INTEREST):
        xin = g[:, i:i + 1] * emb
        h = (xin @ p['g%da_W' % i]).reshape(n, HEADS, HID)
        asrc_l.append(jnp.sum(h * p['g%da_as' % i], axis=-1))
        adst_l.append(jnp.sum(h * p['g%da_ad' % i], axis=-1))
        ha_list.append(h.reshape(n, HEADS * HID))
    b_a = jnp.concatenate([p['g%da_b' % i] for i in range(INTEREST)])
    h1 = jax.nn.relu(_gat_batched(
        jnp.concatenate(ha_list, axis=1), src, dst, emask_all,
        jnp.concatenate(asrc_l, axis=1), jnp.concatenate(adst_l, axis=1),
        b_a, HEADS, HID))

    hb_list, bsrc_l, bdst_l = [], [], []
    for i in range(INTEREST):
        hb = (h1[:, i * HEADS * HID:(i + 1) * HEADS * HID] @ p['g%db_W' % i]).reshape(n, 1, HID)
        bsrc_l.append(jnp.sum(hb * p['g%db_as' % i], axis=-1))
        bdst_l.append(jnp.sum(hb * p['g%db_ad' % i], axis=-1))
        hb_list.append(hb.reshape(n, HID))
    b_b = jnp.concatenate([p['g%db_b' % i] for i in range(INTEREST)])
    sess = _gat_batched(
        jnp.concatenate(hb_list, axis=1), src, dst, emask_all,
        jnp.concatenate(bsrc_l, axis=1), jnp.concatenate(bdst_l, axis=1),
        b_b, 1, HID)
    sections = jnp.bincount(batch, length=B)
    last_idx = jnp.cumsum(sections) - 1
    v_n = sess[last_idx]
    q1 = v_n[batch] @ p['W1'] + p['b1']
    q2 = sess @ p['W2'] + p['b2']
    a = jax.nn.sigmoid(q1 + q2) @ p['qw'] + p['qb']
    s_g = jax.ops.segment_sum(a * sess, batch, num_segments=B)
    s_h = jnp.concatenate([v_n, s_g], axis=1) @ p['W3'] + p['b3']
    sh3 = s_h.reshape(B, INTEREST, HID).transpose(1, 0, 2)
    return _zmax(sh3, p['emb'])


# speedup vs baseline: 3.5108x; 1.0193x over previous
import functools

import jax
import jax.numpy as jnp
from jax.experimental import pallas as pl
from jax.experimental.pallas import tpu as pltpu

HID = 64
HEADS = 16
INTEREST = 3
N_NODE = 100000
B = 256
NEG = 0.2
H3 = INTEREST * HID

ZBLK = 2048


def _zmax_body(sh_ref, emb_ref, out_ref):
    e = emb_ref[...]
    dn = (((1,), (1,)), ((), ()))
    z0 = jax.lax.dot_general(sh_ref[0], e, dn, preferred_element_type=jnp.float32)
    z1 = jax.lax.dot_general(sh_ref[1], e, dn, preferred_element_type=jnp.float32)
    z2 = jax.lax.dot_general(sh_ref[2], e, dn, preferred_element_type=jnp.float32)
    out_ref[...] = jnp.maximum(jnp.maximum(z0, z1), z2)


def _zmax(sh3, emb):
    grid = (pl.cdiv(N_NODE, ZBLK),)
    return pl.pallas_call(
        _zmax_body,
        grid=grid,
        in_specs=[
            pl.BlockSpec((INTEREST, B, HID), lambda j: (0, 0, 0)),
            pl.BlockSpec((ZBLK, HID), lambda j: (j, 0)),
        ],
        out_specs=pl.BlockSpec((B, ZBLK), lambda j: (0, j)),
        out_shape=jax.ShapeDtypeStruct((B, N_NODE), jnp.float32),
    )(sh3, emb)


def _gat_batched(h_all, src, dst, emask_all, asrc_all, adst_all, b_all, heads, outc):
    n = h_all.shape[0]
    emask = jnp.repeat(emask_all, heads, axis=1)
    logit = asrc_all[src] + adst_all[dst]
    logit = jnp.where(logit >= 0, logit, NEG * logit)
    logit = jnp.where(emask, logit, -1e9)
    m = jax.ops.segment_max(logit, dst, num_segments=n)
    pexp = jnp.exp(logit - m[dst])
    pexp = jnp.where(emask, pexp, 0.0)
    den = jax.ops.segment_sum(pexp, dst, num_segments=n)
    alpha = pexp / jnp.clip(den[dst], 1e-16, None)
    hs = h_all.astype(jnp.bfloat16)[src].reshape(-1, alpha.shape[1], outc)
    msg = (hs * alpha.astype(jnp.bfloat16)[:, :, None]).reshape(-1, h_all.shape[1])
    out = jax.ops.segment_sum(msg.astype(jnp.float32), dst, num_segments=n)
    return out + b_all


def kernel(x, edge_index, batch, edge_attr, params):
    p = params
    xi = x - 1
    emb = p['emb'][xi]
    g = jax.nn.softmax((emb @ p['w_int'].T) / 0.1, axis=-1)
    mask = (g > 1.0 / INTEREST).T
    src0 = edge_index[0]
    dst0 = edge_index[1]
    n = emb.shape[0]
    loop = jnp.arange(n)
    src = jnp.concatenate([src0, loop])
    dst = jnp.concatenate([dst0, loop])
    notself = src0 != dst0
    ones_n = jnp.ones((n,), dtype=bool)
    emask_all = jnp.stack(
        [jnp.concatenate([mask[i][src0] & mask[i][dst0] & notself, ones_n])
         for i in range(INTEREST)], axis=1)

    ha_list, asrc_l, adst_l = [], [], []
    for i in range(INTEREST):
        xin = g[:, i:i + 1] * emb
        h = (xin @ p['g%da_W' % i]).reshape(n, HEADS, HID)
        asrc_l.append(jnp.sum(h * p['g%da_as' % i], axis=-1))
        adst_l.append(jnp.sum(h * p['g%da_ad' % i], axis=-1))
        ha_list.append(h.reshape(n, HEADS * HID))
    b_a = jnp.concatenate([p['g%da_b' % i] for i in range(INTEREST)])
    h1 = jax.nn.relu(_gat_batched(
        jnp.concatenate(ha_list, axis=1), src, dst, emask_all,
        jnp.concatenate(asrc_l, axis=1), jnp.concatenate(adst_l, axis=1),
        b_a, HEADS, HID))

    hb_list, bsrc_l, bdst_l = [], [], []
    for i in range(INTEREST):
        hb = (h1[:, i * HEADS * HID:(i + 1) * HEADS * HID] @ p['g%db_W' % i]).reshape(n, 1, HID)
        bsrc_l.append(jnp.sum(hb * p['g%db_as' % i], axis=-1))
        bdst_l.append(jnp.sum(hb * p['g%db_ad' % i], axis=-1))
        hb_list.append(hb.reshape(n, HID))
    b_b = jnp.concatenate([p['g%db_b' % i] for i in range(INTEREST)])
    sess = _gat_batched(
        jnp.concatenate(hb_list, axis=1), src, dst, emask_all,
        jnp.concatenate(bsrc_l, axis=1), jnp.concatenate(bdst_l, axis=1),
        b_b, 1, HID)
    sections = jnp.bincount(batch, length=B)
    last_idx = jnp.cumsum(sections) - 1
    v_n = sess[last_idx]
    q1 = v_n[batch] @ p['W1'] + p['b1']
    q2 = sess @ p['W2'] + p['b2']
    a = jax.nn.sigmoid(q1 + q2) @ p['qw'] + p['qb']
    s_g = jax.ops.segment_sum(a * sess, batch, num_segments=B)
    s_h = jnp.concatenate([v_n, s_g], axis=1) @ p['W3'] + p['b3']
    sh3 = s_h.reshape(B, INTEREST, HID).transpose(1, 0, 2)
    return _zmax(sh3, p['emb'])
